# baseline (speedup 1.0000x reference)
import functools

import jax
import jax.numpy as jnp
from jax import lax
from jax.experimental import pallas as pl
from jax.experimental.pallas import tpu as pltpu
from jax.experimental.pallas import tpu_sc as plsc

BS, NSEQ, N, NCXT = 64, 32, 4096, 32
TOT = N * NCXT


_BB = 8


def _son_map_body(gsub_ref, subm_ref, gobj_ref, objm_ref, attn_obj_ref, out_ref):
    subm = subm_ref[...]
    objm = objm_ref[...]
    son = jnp.sum(gsub_ref[...] * subm[:, :, None], axis=1)
    son = son + jnp.sum(gobj_ref[...] * objm[:, :, None], axis=1)
    num_rel = jnp.sum(subm, axis=1, keepdims=True) + jnp.sum(objm, axis=1, keepdims=True)
    norm = jnp.max(son, axis=1, keepdims=True)
    norm = jnp.where(norm <= 1.0, 1.0, norm)
    son = son / norm
    out_ref[...] = jnp.where(num_rel == 0.0, attn_obj_ref[...], son)


def _son_map_tc(gsub, subm_f, gobj, objm_f, attn_obj):
    grid = BS // _BB
    return pl.pallas_call(
        _son_map_body,
        grid=(grid,),
        in_specs=[
            pl.BlockSpec((_BB, NSEQ, N), lambda b: (b, 0, 0)),
            pl.BlockSpec((_BB, NSEQ), lambda b: (b, 0)),
            pl.BlockSpec((_BB, NSEQ, N), lambda b: (b, 0, 0)),
            pl.BlockSpec((_BB, NSEQ), lambda b: (b, 0)),
            pl.BlockSpec((_BB, N), lambda b: (b, 0)),
        ],
        out_specs=pl.BlockSpec((_BB, N), lambda b: (b, 0)),
        out_shape=jax.ShapeDtypeStruct((BS, N), jnp.float32),
    )(gsub, subm_f, gobj, objm_f, attn_obj)


_NW = 32
_BPW = BS // _NW
_NG = N // 16


def _gather_attn_sc(son_map, ind_t, rel_t):
    mesh = plsc.VectorSubcoreMesh(core_axis_name="c", subcore_axis_name="s")

    @functools.partial(
        pl.kernel,
        mesh=mesh,
        out_type=jax.ShapeDtypeStruct((BS, N), jnp.float32),
        compiler_params=pltpu.CompilerParams(needs_layout_passes=False),
        scratch_types=[
            pltpu.VMEM((N,), jnp.float32),
            pltpu.VMEM((N,), jnp.int32),
            pltpu.VMEM((N,), jnp.int32),
            pltpu.VMEM((N,), jnp.float32),
            pltpu.VMEM((N,), jnp.float32),
            pltpu.VMEM((N,), jnp.float32),
            pltpu.SemaphoreType.DMA,
            pltpu.SemaphoreType.DMA,
            pltpu.SemaphoreType.DMA,
            pltpu.SemaphoreType.DMA,
        ],
    )
    def sc_kernel(son_hbm, ind_hbm, rel_hbm, out_hbm, table_v, ind_v0, ind_v1,
                  rel_v0, rel_v1, acc_v, si0, si1, sr0, sr1):
        wid = lax.axis_index("s") * 2 + lax.axis_index("c")
        ind_bufs = (ind_v0, ind_v1)
        rel_bufs = (rel_v0, rel_v1)
        sems = ((si0, sr0), (si1, sr1))

        def start_row(b, c, buf):
            ci = pltpu.async_copy(ind_hbm.at[b, c], ind_bufs[buf], sems[buf][0])
            cr = pltpu.async_copy(rel_hbm.at[b, c], rel_bufs[buf], sems[buf][1])
            return ci, cr

        for k in range(_BPW):
            b = wid * _BPW + k
            pltpu.sync_copy(son_hbm.at[b], table_v)
            copies = start_row(b, 0, 0)
            for c in range(NCXT):
                buf = c % 2
                copies[0].wait()
                copies[1].wait()
                if c + 1 < NCXT:
                    copies = start_row(b, c + 1, 1 - buf)
                iv_ref, rv_ref = ind_bufs[buf], rel_bufs[buf]
                if c == 0:
                    @plsc.parallel_loop(0, _NG, unroll=8)
                    def init_body(g):
                        iv = iv_ref[pl.ds(g * 16, 16)]
                        rv = rv_ref[pl.ds(g * 16, 16)]
                        tv = plsc.load_gather(table_v, [iv])
                        acc_v[pl.ds(g * 16, 16)] = rv * tv
                else:
                    @plsc.parallel_loop(0, _NG, unroll=8)
                    def add_body(g):
                        iv = iv_ref[pl.ds(g * 16, 16)]
                        rv = rv_ref[pl.ds(g * 16, 16)]
                        tv = plsc.load_gather(table_v, [iv])
                        plsc.addupdate(acc_v.at[pl.ds(g * 16, 16)], rv * tv)

            @plsc.parallel_loop(0, _NG, unroll=4,
                                carry=jnp.full((16,), -3.0e38, jnp.float32))
            def max_body(g, mx):
                return jnp.maximum(mx, acc_v[pl.ds(g * 16, 16)])

            row_max = lax.reduce_max(max_body, (0,))
            norm = jnp.where(row_max <= 1.0, 1.0, row_max)
            inv_v = jnp.ones((16,), jnp.float32) / jnp.broadcast_to(norm, (16,))

            @plsc.parallel_loop(0, _NG, unroll=8)
            def scale_body(g):
                acc_v[pl.ds(g * 16, 16)] = acc_v[pl.ds(g * 16, 16)] * inv_v

            pltpu.sync_copy(acc_v, out_hbm.at[b])

    return sc_kernel(son_map, ind_t, rel_t)


def kernel(attn_relation, relation_ind, global_sub_attn_maps, sub_mask,
           global_obj_attn_maps, obj_mask, attn_obj):
    subm_f = sub_mask.astype(jnp.float32)
    objm_f = obj_mask.astype(jnp.float32)
    son_map = _son_map_tc(global_sub_attn_maps, subm_f,
                          global_obj_attn_maps, objm_f, attn_obj)
    ind_t = jnp.swapaxes(relation_ind, 1, 2)
    rel_t = jnp.swapaxes(attn_relation, 1, 2)
    attn = _gather_attn_sc(son_map, ind_t, rel_t)
    return (attn, son_map)


# speedup vs baseline: 1.0550x; 1.0550x over previous
import functools

import jax
import jax.numpy as jnp
from jax import lax
from jax.experimental import pallas as pl
from jax.experimental.pallas import tpu as pltpu
from jax.experimental.pallas import tpu_sc as plsc

BS, NSEQ, N, NCXT = 64, 32, 4096, 32
TOT = N * NCXT


_BB = 8


def _son_map_body(gsub_ref, subm_ref, gobj_ref, objm_ref, attn_obj_ref, out_ref):
    subm = subm_ref[...]
    objm = objm_ref[...]
    son = jnp.sum(gsub_ref[...] * subm[:, :, None], axis=1)
    son = son + jnp.sum(gobj_ref[...] * objm[:, :, None], axis=1)
    num_rel = jnp.sum(subm, axis=1, keepdims=True) + jnp.sum(objm, axis=1, keepdims=True)
    norm = jnp.max(son, axis=1, keepdims=True)
    norm = jnp.where(norm <= 1.0, 1.0, norm)
    son = son / norm
    out_ref[...] = jnp.where(num_rel == 0.0, attn_obj_ref[...], son)


def _son_map_tc(gsub, subm_f, gobj, objm_f, attn_obj):
    grid = BS // _BB
    return pl.pallas_call(
        _son_map_body,
        grid=(grid,),
        in_specs=[
            pl.BlockSpec((_BB, NSEQ, N), lambda b: (b, 0, 0)),
            pl.BlockSpec((_BB, NSEQ), lambda b: (b, 0)),
            pl.BlockSpec((_BB, NSEQ, N), lambda b: (b, 0, 0)),
            pl.BlockSpec((_BB, NSEQ), lambda b: (b, 0)),
            pl.BlockSpec((_BB, N), lambda b: (b, 0)),
        ],
        out_specs=pl.BlockSpec((_BB, N), lambda b: (b, 0)),
        out_shape=jax.ShapeDtypeStruct((BS, N), jnp.float32),
    )(gsub, subm_f, gobj, objm_f, attn_obj)


_NW = 32
_BPW = BS // _NW
_NG = N // 16


def _gather_attn_sc(son_map, ind_t, rel_t):
    mesh = plsc.VectorSubcoreMesh(core_axis_name="c", subcore_axis_name="s")

    @functools.partial(
        pl.kernel,
        mesh=mesh,
        out_type=jax.ShapeDtypeStruct((BS, N), jnp.float32),
        compiler_params=pltpu.CompilerParams(needs_layout_passes=False),
        scratch_types=[
            pltpu.VMEM((N,), jnp.float32),
            pltpu.VMEM((N,), jnp.int32),
            pltpu.VMEM((N,), jnp.int32),
            pltpu.VMEM((N,), jnp.float32),
            pltpu.VMEM((N,), jnp.float32),
            pltpu.VMEM((N,), jnp.float32),
            pltpu.SemaphoreType.DMA,
            pltpu.SemaphoreType.DMA,
            pltpu.SemaphoreType.DMA,
            pltpu.SemaphoreType.DMA,
        ],
    )
    def sc_kernel(son_hbm, ind_hbm, rel_hbm, out_hbm, table_v, ind_v0, ind_v1,
                  rel_v0, rel_v1, acc_v, si0, si1, sr0, sr1):
        wid = lax.axis_index("s") * 2 + lax.axis_index("c")
        ind_bufs = (ind_v0, ind_v1)
        rel_bufs = (rel_v0, rel_v1)
        sems = ((si0, sr0), (si1, sr1))

        def start_row(b, c, buf):
            ci = pltpu.async_copy(ind_hbm.at[b, c], ind_bufs[buf], sems[buf][0])
            cr = pltpu.async_copy(rel_hbm.at[b, c], rel_bufs[buf], sems[buf][1])
            return ci, cr

        for k in range(_BPW):
            b = wid * _BPW + k
            pltpu.sync_copy(son_hbm.at[b], table_v)
            copies = start_row(b, 0, 0)
            for c in range(NCXT):
                buf = c % 2
                nxt = start_row(b, c + 1, 1 - buf) if c + 1 < NCXT else None
                copies[0].wait()
                copies[1].wait()
                if nxt is not None:
                    copies = nxt
                iv_ref, rv_ref = ind_bufs[buf], rel_bufs[buf]
                if c == 0:
                    @plsc.parallel_loop(0, _NG, unroll=4)
                    def init_body(g):
                        iv = iv_ref[pl.ds(g * 16, 16)]
                        rv = rv_ref[pl.ds(g * 16, 16)]
                        tv = plsc.load_gather(table_v, [iv])
                        acc_v[pl.ds(g * 16, 16)] = rv * tv
                else:
                    @plsc.parallel_loop(0, _NG, unroll=4)
                    def add_body(g):
                        iv = iv_ref[pl.ds(g * 16, 16)]
                        rv = rv_ref[pl.ds(g * 16, 16)]
                        tv = plsc.load_gather(table_v, [iv])
                        plsc.addupdate(acc_v.at[pl.ds(g * 16, 16)], rv * tv)

            @plsc.parallel_loop(0, _NG, unroll=4,
                                carry=jnp.full((16,), -3.0e38, jnp.float32))
            def max_body(g, mx):
                return jnp.maximum(mx, acc_v[pl.ds(g * 16, 16)])

            row_max = lax.reduce_max(max_body, (0,))
            norm = jnp.where(row_max <= 1.0, 1.0, row_max)
            inv_v = jnp.ones((16,), jnp.float32) / jnp.broadcast_to(norm, (16,))

            @plsc.parallel_loop(0, _NG, unroll=4)
            def scale_body(g):
                acc_v[pl.ds(g * 16, 16)] = acc_v[pl.ds(g * 16, 16)] * inv_v

            pltpu.sync_copy(acc_v, out_hbm.at[b])

    return sc_kernel(son_map, ind_t, rel_t)


def kernel(attn_relation, relation_ind, global_sub_attn_maps, sub_mask,
           global_obj_attn_maps, obj_mask, attn_obj):
    subm_f = sub_mask.astype(jnp.float32)
    objm_f = obj_mask.astype(jnp.float32)
    son_map = _son_map_tc(global_sub_attn_maps, subm_f,
                          global_obj_attn_maps, objm_f, attn_obj)
    ind_t = jnp.swapaxes(relation_ind, 1, 2)
    rel_t = jnp.swapaxes(attn_relation, 1, 2)
    attn = _gather_attn_sc(son_map, ind_t, rel_t)
    return (attn, son_map)
